# Initial kernel scaffold; baseline (speedup 1.0000x reference)
import jax
import jax.numpy as jnp
from jax.experimental import pallas as pl


def kernel(x, edge_index, W1, b1, W2, b2, Wt, bt, Ws, bs, Wf, bf, Wa, ba):
    raise NotImplementedError("write your pallas kernel here")


# speedup vs baseline: 18.6152x; 18.6152x over previous
import functools

import jax
import jax.numpy as jnp
from jax import lax
from jax.experimental import pallas as pl
from jax.experimental.pallas import tpu as pltpu
from jax.experimental.pallas import tpu_sc as plsc

N = 10000
NP = 10240
D = 128
E = 320000
NC, NS = 2, 16
NW = NC * NS
CH = 128
NCHUNK = 79
EPW = NCHUNK * CH
EPAD = NW * EPW
RPT = NP // NS
HEADS = (10, 20, 15, 100)
HCAT = 256


def _sc_deg_body(dst_hbm, zeros_hbm, ones_hbm, degp_hbm, idx_v, ones_v, acc_sh):
    c = lax.axis_index("c")
    s = lax.axis_index("s")
    wid = s * NC + c
    r0 = s * RPT
    pltpu.sync_copy(zeros_hbm.at[pl.ds(r0, RPT)], acc_sh.at[pl.ds(r0, RPT)])
    pltpu.sync_copy(dst_hbm.at[wid], idx_v)
    pltpu.sync_copy(ones_hbm, ones_v)
    plsc.subcore_barrier()

    def body(j, carry):
        pltpu.sync_copy(ones_v, acc_sh.at[idx_v.at[j]], add=True)
        return carry

    lax.fori_loop(0, NCHUNK, body, 0)
    plsc.subcore_barrier()
    pltpu.sync_copy(acc_sh.at[pl.ds(r0, RPT)], degp_hbm.at[c, pl.ds(r0, RPT)])


def _sc_agg_body(hh_hbm, src_hbm, dst_hbm, zeros_hbm, aggp_hbm,
                 src_v, dst_v, rows_v, sem, acc_sh):
    c = lax.axis_index("c")
    s = lax.axis_index("s")
    wid = s * NC + c
    r0 = s * RPT
    pltpu.sync_copy(zeros_hbm.at[pl.ds(r0, RPT)], acc_sh.at[pl.ds(r0, RPT)])
    pltpu.sync_copy(src_hbm.at[wid], src_v)
    pltpu.sync_copy(dst_hbm.at[wid], dst_v)
    plsc.subcore_barrier()

    def body(j, carry):
        pltpu.async_copy(hh_hbm.at[src_v.at[j]], rows_v, sem).wait()
        pltpu.sync_copy(rows_v, acc_sh.at[dst_v.at[j]], add=True)
        return carry

    lax.fori_loop(0, NCHUNK, body, 0)
    plsc.subcore_barrier()
    pltpu.sync_copy(acc_sh.at[pl.ds(r0, RPT)], aggp_hbm.at[c, pl.ds(r0, RPT)])


@functools.cache
def _sc_kernels():
    mesh = plsc.VectorSubcoreMesh(
        core_axis_name="c", subcore_axis_name="s",
        num_cores=NC, num_subcores=NS)
    sc_deg = pl.kernel(
        _sc_deg_body,
        out_type=jax.ShapeDtypeStruct((NC, NP, D), jnp.float32),
        mesh=mesh,
        scratch_types=[
            pltpu.VMEM((NCHUNK, CH), jnp.int32),
            pltpu.VMEM((CH, D), jnp.float32),
            pltpu.VMEM_SHARED((NP, D), jnp.float32),
        ])
    sc_agg = pl.kernel(
        _sc_agg_body,
        out_type=jax.ShapeDtypeStruct((NC, NP, D), jnp.float32),
        mesh=mesh,
        scratch_types=[
            pltpu.VMEM((NCHUNK, CH), jnp.int32),
            pltpu.VMEM((NCHUNK, CH), jnp.int32),
            pltpu.VMEM((CH, D), jnp.float32),
            pltpu.SemaphoreType.DMA,
            pltpu.VMEM_SHARED((NP, D), jnp.float32),
        ])
    return sc_deg, sc_agg


R = 512
GRID = NP // R


def _dinv_of(degp):
    deg = degp[0, :, 0:1] + degp[1, :, 0:1] + 1.0
    return lax.rsqrt(deg)


def _tc1_body(x_ref, degp_ref, w_ref, hh_ref):
    dinv = _dinv_of(degp_ref[...])
    hh_ref[...] = dinv * jnp.dot(x_ref[...], w_ref[...],
                                 preferred_element_type=jnp.float32)


def _tc2_body(aggp_ref, hh1_ref, degp_ref, b_ref, w_ref, hh2_ref):
    dinv = _dinv_of(degp_ref[...])
    a = aggp_ref[...]
    z = jax.nn.relu(dinv * (a[0] + a[1] + hh1_ref[...]) + b_ref[...])
    hh2_ref[...] = dinv * jnp.dot(z, w_ref[...],
                                  preferred_element_type=jnp.float32)


def _tc3_body(aggp_ref, hh2_ref, degp_ref, b_ref, w_ref, bc_ref, out_ref):
    dinv = _dinv_of(degp_ref[...])
    a = aggp_ref[...]
    g = dinv * (a[0] + a[1] + hh2_ref[...]) + b_ref[...]
    out_ref[...] = jnp.dot(g, w_ref[...],
                           preferred_element_type=jnp.float32) + bc_ref[...]


_row_spec = pl.BlockSpec((R, D), lambda i: (i, 0))
_degp_spec = pl.BlockSpec((2, R, D), lambda i: (0, i, 0))
_aggp_spec = pl.BlockSpec((2, R, D), lambda i: (0, i, 0))
_w_spec = pl.BlockSpec((D, D), lambda i: (0, 0))
_b_spec = pl.BlockSpec((1, D), lambda i: (0, 0))

_tc1 = pl.pallas_call(
    _tc1_body, grid=(GRID,),
    in_specs=[_row_spec, _degp_spec, _w_spec],
    out_specs=_row_spec,
    out_shape=jax.ShapeDtypeStruct((NP, D), jnp.float32))

_tc2 = pl.pallas_call(
    _tc2_body, grid=(GRID,),
    in_specs=[_aggp_spec, _row_spec, _degp_spec, _b_spec, _w_spec],
    out_specs=_row_spec,
    out_shape=jax.ShapeDtypeStruct((NP, D), jnp.float32))

_tc3 = pl.pallas_call(
    _tc3_body, grid=(GRID,),
    in_specs=[_aggp_spec, _row_spec, _degp_spec, _b_spec,
              pl.BlockSpec((D, HCAT), lambda i: (0, 0)),
              pl.BlockSpec((1, HCAT), lambda i: (0, 0))],
    out_specs=pl.BlockSpec((R, HCAT), lambda i: (i, 0)),
    out_shape=jax.ShapeDtypeStruct((NP, HCAT), jnp.float32))


def kernel(x, edge_index, W1, b1, W2, b2, Wt, bt, Ws, bs, Wf, bf, Wa, ba):
    src = edge_index[0].astype(jnp.int32)
    dst = edge_index[1].astype(jnp.int32)
    pad_ids = N + (jnp.arange(EPAD - E, dtype=jnp.int32) % 32)
    src_p = jnp.concatenate([src, pad_ids]).reshape(NW, NCHUNK, CH)
    dst_p = jnp.concatenate([dst, pad_ids]).reshape(NW, NCHUNK, CH)

    zerosD = jnp.zeros((NP, D), jnp.float32)
    onesD = jnp.ones((CH, D), jnp.float32)
    xp = jnp.pad(x, ((0, NP - N), (0, 0)))
    b1r = b1.reshape(1, D)
    b2r = b2.reshape(1, D)
    Wcat = jnp.pad(jnp.concatenate([Wt, Ws, Wf, Wa], axis=1),
                   ((0, 0), (0, HCAT - sum(HEADS))))
    bcat = jnp.pad(jnp.concatenate([bt, bs, bf, ba]),
                   (0, HCAT - sum(HEADS))).reshape(1, HCAT)

    _sc_deg, _sc_agg = _sc_kernels()
    degp = _sc_deg(dst_p, zerosD, onesD)
    hh1 = _tc1(xp, degp, W1)
    aggp1 = _sc_agg(hh1, src_p, dst_p, zerosD)
    hh2 = _tc2(aggp1, hh1, degp, b1r, W2)
    aggp2 = _sc_agg(hh2, src_p, dst_p, zerosD)
    outc = _tc3(aggp2, hh2, degp, b2r, Wcat, bcat)

    o = outc[:N]
    c0, c1, c2, c3 = 0, 10, 30, 45
    return (o[:, c0:c1], o[:, c1:c2], o[:, c2:c3], o[:, c3:c3 + 100])
